# baseline (speedup 1.0000x reference)
import functools
import math

import jax
import jax.numpy as jnp
from jax import lax
from jax.experimental import pallas as pl
from jax.experimental.pallas import tpu as pltpu
from jax.experimental.pallas import tpu_sc as plsc

D_MODEL = 64
SCALE = math.sqrt(D_MODEL)
NC, NS = 2, 16
NW = NC * NS
LANES = 16
VOCAB_N = 1000000

C = 128
NBUF = 2
SSTR = 65

CD = 256
NCH_D = (VOCAB_N + CD - 1) // CD
D_ITER = (NCH_D + NW - 1) // NW

B0, B1 = 4096, 200
TOTAL = B0 * B1
CH_PER_W = TOTAL // C // NW
QPB = B0 // C

_MESH = dict(core_axis_name="c", subcore_axis_name="s")


@jax.jit
def _run(x_t, table):
    @functools.partial(
        pl.kernel,
        out_type=jax.ShapeDtypeStruct((VOCAB_N * D_MODEL,), jnp.float32),
        mesh=plsc.VectorSubcoreMesh(**_MESH),
        scratch_types=[
            pltpu.VMEM((CD, D_MODEL), jnp.float32),
            pltpu.VMEM((CD * D_MODEL,), jnp.float32),
            pltpu.SemaphoreType.DMA,
        ],
        compiler_params=pltpu.CompilerParams(
            use_tc_tiling_on_sc=True, needs_layout_passes=False
        ),
    )
    def _depad(tab_hbm, out_hbm, vin, vout, sem):
        wid = lax.axis_index("s") * NC + lax.axis_index("c")

        def step(g, carry):
            gc = wid + g * NW

            @pl.when(gc < NCH_D)
            def _():
                start = jnp.minimum(gc * CD, VOCAB_N - CD)
                start = pl.multiple_of((start // 8) * 8, 8)
                pltpu.sync_copy(tab_hbm.at[pl.ds(start, CD)], vin)

                def row(r, c2):
                    for j in range(D_MODEL // LANES):
                        vout[pl.ds(r * D_MODEL + j * LANES, LANES)] = (
                            vin[r, pl.ds(j * LANES, LANES)]
                        )
                    return c2

                lax.fori_loop(0, CD, row, 0)
                pltpu.sync_copy(
                    vout, out_hbm.at[pl.ds(start * D_MODEL, CD * D_MODEL)]
                )

            return carry

        lax.fori_loop(0, D_ITER, step, 0)

    @functools.partial(
        pl.kernel,
        out_type=jax.ShapeDtypeStruct((B1, 8, B0 // 128, 8, 128), jnp.float32),
        mesh=plsc.VectorSubcoreMesh(**_MESH),
        scratch_types=(
            [pltpu.VMEM((C,), jnp.int32) for _ in range(NBUF)]
            + [pltpu.VMEM((C, D_MODEL), jnp.float32) for _ in range(NBUF)]
            + [pltpu.VMEM((C * SSTR,), jnp.float32) for _ in range(NBUF)]
            + [pltpu.VMEM((8, 1, 8, 128), jnp.float32) for _ in range(NBUF)]
            + [pltpu.SemaphoreType.DMA for _ in range(2 * NBUF)]
        ),
        compiler_params=pltpu.CompilerParams(
            use_tc_tiling_on_sc=False, needs_layout_passes=False
        ),
    )
    def _emb_lookup(x_hbm, table_hbm, out_hbm, *scratch):
        idx = scratch[:NBUF]
        rows = scratch[NBUF:2 * NBUF]
        stg = scratch[2 * NBUF:3 * NBUF]
        tbuf = scratch[3 * NBUF:4 * NBUF]
        gsem = scratch[4 * NBUF:5 * NBUF]
        osem = scratch[5 * NBUF:]

        wid = lax.axis_index("s") * NC + lax.axis_index("c")
        chunk0 = wid * CH_PER_W
        iota = lax.iota(jnp.int32, LANES)

        def start_gather(b, gc):
            start = pl.multiple_of(gc * C, C)
            pltpu.sync_copy(x_hbm.at[pl.ds(start, C)], idx[b])
            pltpu.async_copy(table_hbm.at[idx[b]], rows[b], gsem[b])

        def wait_gather(b):
            pltpu.make_async_copy(table_hbm.at[idx[b]], rows[b], gsem[b]).wait()

        def reformat(b):
            def s1(r, carry):
                for jj in range(D_MODEL // LANES):
                    stg[b][pl.ds(r * SSTR + jj * LANES, LANES)] = (
                        rows[b][r, pl.ds(jj * LANES, LANES)]
                    )
                return carry

            lax.fori_loop(0, C, s1, 0)

            def s2(j16, carry):
                addr0 = (j16 * LANES + iota) * SSTR
                for d in range(D_MODEL):
                    v = plsc.load_gather(stg[b], [addr0 + d])
                    tbuf[b][d // 8, 0, d % 8, pl.ds(j16 * LANES, LANES)] = v * SCALE
                return carry

            lax.fori_loop(0, C // LANES, s2, 0)

        def write_out(b, b1, q):
            for db in range(8):
                pltpu.async_copy(
                    tbuf[b].at[db], out_hbm.at[b1, db, pl.ds(q, 1)], osem[b]
                )

        def wait_writes(b, b1, q):
            for db in range(8):
                pltpu.make_async_copy(
                    tbuf[b].at[db], out_hbm.at[b1, db, pl.ds(q, 1)], osem[b]
                ).wait()

        def process(b, gc, do_wait_writes):
            b1 = gc // QPB
            q = gc % QPB
            wait_gather(b)
            if do_wait_writes:
                wait_writes(b, b1, q)
            reformat(b)
            write_out(b, b1, q)

        for b in range(NBUF):
            start_gather(b, chunk0 + b)
        for b in range(NBUF):
            process(b, chunk0 + b, do_wait_writes=False)
            start_gather(b, chunk0 + b + NBUF)

        def main(i, carry):
            for b in range(NBUF):
                gc = chunk0 + i * NBUF + b
                process(b, gc, do_wait_writes=True)
                start_gather(b, gc + NBUF)
            return carry

        lax.fori_loop(1, CH_PER_W // NBUF - 1, main, 0)

        for b in range(NBUF):
            gc = chunk0 + CH_PER_W - NBUF + b
            process(b, gc, do_wait_writes=True)
        for b in range(NBUF):
            gc = chunk0 + CH_PER_W - NBUF + b
            wait_writes(b, gc // QPB, gc % QPB)

    t_lin = _depad(table)
    return _emb_lookup(x_t, t_lin.reshape(VOCAB_N, D_MODEL))


def kernel(x, table):
    x_t = x.T.reshape(TOTAL)
    out5 = _run(x_t, table)
    return out5.transpose(2, 4, 0, 1, 3).reshape(B0, B1, D_MODEL)


# speedup vs baseline: 1.7710x; 1.7710x over previous
import functools
import math

import jax
import jax.numpy as jnp
from jax import lax
from jax.experimental import pallas as pl
from jax.experimental.pallas import tpu as pltpu
from jax.experimental.pallas import tpu_sc as plsc

D_MODEL = 64
SCALE = math.sqrt(D_MODEL)
NC, NS = 2, 16
NW = NC * NS
CHUNK = 512
NBUF = 2
LANES = 16


@functools.partial(jax.jit, static_argnames=("total",))
def _emb_lookup(x_flat, table, *, total):
    b_per_w = total // NW
    nchunks = b_per_w // CHUNK
    assert nchunks % NBUF == 0 and nchunks // NBUF >= 2

    mesh = plsc.VectorSubcoreMesh(core_axis_name="c", subcore_axis_name="s")

    @functools.partial(
        pl.kernel,
        out_type=jax.ShapeDtypeStruct((total, D_MODEL), jnp.float32),
        mesh=mesh,
        scratch_types=(
            [pltpu.VMEM((CHUNK,), jnp.int32) for _ in range(NBUF)]
            + [pltpu.VMEM((CHUNK, D_MODEL), jnp.float32) for _ in range(NBUF)]
            + [pltpu.SemaphoreType.DMA for _ in range(2 * NBUF)]
        ),
        compiler_params=pltpu.CompilerParams(use_tc_tiling_on_sc=False),
    )
    def body(x_hbm, table_hbm, out_hbm, *scratch):
        idx = scratch[:NBUF]
        rows = scratch[NBUF:2 * NBUF]
        gsem = scratch[2 * NBUF:3 * NBUF]
        osem = scratch[3 * NBUF:]

        wid = lax.axis_index("s") * NC + lax.axis_index("c")
        base = pl.multiple_of(wid * b_per_w, CHUNK)

        def start_gather(b, g):
            start = pl.multiple_of(base + g * CHUNK, CHUNK)
            pltpu.sync_copy(x_hbm.at[pl.ds(start, CHUNK)], idx[b])
            pltpu.async_copy(table_hbm.at[idx[b]], rows[b], gsem[b])

        def wait_gather(b):
            pltpu.make_async_copy(table_hbm.at[idx[b]], rows[b], gsem[b]).wait()

        def scale_buf(b):
            def row_body(r, carry):
                for j in range(D_MODEL // LANES):
                    sl = pl.ds(j * LANES, LANES)
                    rows[b][r, sl] = rows[b][r, sl] * SCALE
                return carry
            lax.fori_loop(0, CHUNK, row_body, 0, unroll=2)

        def start_write(b, g):
            start = pl.multiple_of(base + g * CHUNK, CHUNK)
            pltpu.async_copy(rows[b], out_hbm.at[pl.ds(start, CHUNK)], osem[b])

        def wait_write(b, g):
            start = pl.multiple_of(base + g * CHUNK, CHUNK)
            pltpu.make_async_copy(
                rows[b], out_hbm.at[pl.ds(start, CHUNK)], osem[b]
            ).wait()

        for b in range(NBUF):
            start_gather(b, b)

        def main(i, carry):
            for b in range(NBUF):
                g = i * NBUF + b
                wait_gather(b)
                scale_buf(b)
                start_write(b, g)
                wait_write(b, g)
                start_gather(b, g + NBUF)
            return carry

        lax.fori_loop(0, nchunks // NBUF - 1, main, 0)

        for b in range(NBUF):
            g = nchunks - NBUF + b
            wait_gather(b)
            scale_buf(b)
            start_write(b, g)
            wait_write(b, g)

    return body(x_flat, table)


def kernel(x, table):
    total = x.shape[0] * x.shape[1]
    out = _emb_lookup(x.reshape(total), table, total=total)
    return out.reshape(x.shape[0], x.shape[1], D_MODEL)
